# baseline (speedup 1.0000x reference)
import jax
import jax.numpy as jnp
from jax import lax
from jax.experimental import pallas as pl
from jax.experimental.pallas import tpu as pltpu
from jax.experimental.pallas import tpu_sc as plsc

N = 10000
E = 320000
FN = 64
IN = 64
HID = 128
FE = 9
LAT = 128

NC = 2
NS = 16
LANES = 16
HALF = N // NC
EPB = E // NS
SEGSZ = 2000
NSEG = EPB // SEGSZ
SEGG = SEGSZ // LANES
K = 96
NBUF = 2
LCAP = SEGSZ + K
APAD = 5120
RPT = APAD // NS
BE = 8000


def _prep_body(fn_ref, hn_ref, Wn1, Wn2, bn, W1, W2, W3, Wep, be, bm,
               h_ref, A_ref, B_ref, Wp_ref):
    h = (jnp.dot(fn_ref[...], Wn1[...], preferred_element_type=jnp.float32)
         + jnp.dot(hn_ref[...], Wn2[...], preferred_element_type=jnp.float32)
         + bn[...])
    h_ref[...] = h
    A_ref[...] = jnp.dot(h, W1[...], preferred_element_type=jnp.float32)
    c0 = jnp.dot(be[...], W3[...], preferred_element_type=jnp.float32) + bm[...]
    B_ref[...] = jnp.dot(h, W2[...], preferred_element_type=jnp.float32) + c0
    Wp_ref[...] = jnp.dot(Wep[...], W3[...], preferred_element_type=jnp.float32)


_prep_call = pl.pallas_call(
    _prep_body,
    out_shape=[
        jax.ShapeDtypeStruct((N, HID), jnp.float32),
        jax.ShapeDtypeStruct((N, HID), jnp.float32),
        jax.ShapeDtypeStruct((N, HID), jnp.float32),
        jax.ShapeDtypeStruct((16, HID), jnp.float32),
    ],
)


def _edgec_body(fe_ref, Wp_ref, C_ref):
    C_ref[...] = jnp.dot(fe_ref[...], Wp_ref[...],
                         preferred_element_type=jnp.float32)


_edgec_call = pl.pallas_call(
    _edgec_body,
    grid=(E // BE,),
    in_specs=[
        pl.BlockSpec((BE, 16), lambda i: (i, 0)),
        pl.BlockSpec((16, HID), lambda i: (0, 0)),
    ],
    out_specs=pl.BlockSpec((BE, HID), lambda i: (i, 0)),
    out_shape=jax.ShapeDtypeStruct((E, HID), jnp.float32),
)


def _sc_edge_body(A_hbm, B_hbm, C_hbm, src_hbm, dst_hbm, z_hbm, out_hbm,
                  raw_s, raw_d, ls, ld, le,
                  srcb0, dstg0, dstl0, eidb0, bufA0, bufB0, bufC0,
                  srcb1, dstg1, dstl1, eidb1, bufA1, bufB1, bufC1,
                  acc_sh,
                  semA0, semB0, semC0, ssem0,
                  semA1, semB1, semC1, ssem1):
    srcb = (srcb0, srcb1)
    dstg = (dstg0, dstg1)
    dstl = (dstl0, dstl1)
    eidb = (eidb0, eidb1)
    bufA = (bufA0, bufA1)
    bufB = (bufB0, bufB1)
    bufC = (bufC0, bufC1)
    semA = (semA0, semA1)
    semB = (semB0, semB1)
    semC = (semC0, semC1)
    ssem = (ssem0, ssem1)
    cid = lax.axis_index("c")
    sid = lax.axis_index("s")
    lo = cid * HALF

    pltpu.sync_copy(z_hbm.at[pl.ds(sid * RPT, RPT)],
                    acc_sh.at[pl.ds(sid * RPT, RPT)])
    plsc.subcore_barrier()

    iota = lax.iota(jnp.int32, LANES)
    lo_v = jnp.full((LANES,), lo, jnp.int32)
    ones_v = jnp.full((LANES,), 1, jnp.int32)
    zeros_v = jnp.full((LANES,), 0, jnp.int32)

    def segment(seg, carry0):
        sbase = sid * EPB + seg * SEGSZ
        pltpu.sync_copy(src_hbm.at[pl.ds(sbase, SEGSZ)], raw_s)
        pltpu.sync_copy(dst_hbm.at[pl.ds(sbase, SEGSZ)], raw_d)

        def compact(g, off_v):
            s = raw_s[pl.ds(g * LANES, LANES)]
            d = raw_d[pl.ds(g * LANES, LANES)]
            rel = d - lo_v
            lm = rel.astype(jnp.uint32) < jnp.uint32(HALF)
            lmi = jnp.where(lm, ones_v, zeros_v)
            excl = plsc.cumsum(lmi) - lmi
            idxv = excl + off_v
            plsc.store_scatter(ls, [idxv], s, mask=lm)
            plsc.store_scatter(ld, [idxv], rel, mask=lm)
            plsc.store_scatter(le, [idxv],
                               jnp.full((LANES,), sbase + g * LANES,
                                        jnp.int32) + iota, mask=lm)
            return off_v + plsc.all_reduce_population_count(lm)

        cnt_v = lax.fori_loop(0, SEGG, compact, zeros_v)

        for j in range(K // LANES):
            idxv = cnt_v + jnp.full((LANES,), j * LANES, jnp.int32) + iota
            plsc.store_scatter(ls, [idxv], zeros_v)
            plsc.store_scatter(ld, [idxv],
                               jnp.full((LANES,), APAD - 1, jnp.int32))
            plsc.store_scatter(le, [idxv], zeros_v)

        def valid(i):
            return jnp.any(jnp.full((LANES,), i * K, jnp.int32) < cnt_v)

        def scat_wait(j):
            pltpu.make_async_copy(bufA[j], acc_sh.at[dstl[j]],
                                  ssem[j]).wait()

        def fire(i, j):
            for jj in range(K // LANES):
                sl = pl.ds(i * K + jj * LANES, LANES)
                t = pl.ds(jj * LANES, LANES)
                srcb[j][t] = ls[sl]
                v = ld[sl]
                dstl[j][t] = v
                dstg[j][t] = jnp.minimum(v, HALF - 1) + lo_v
                eidb[j][t] = le[sl]
            pltpu.async_copy(A_hbm.at[srcb[j]], bufA[j], semA[j])
            pltpu.async_copy(B_hbm.at[dstg[j]], bufB[j], semB[j])
            pltpu.async_copy(C_hbm.at[eidb[j]], bufC[j], semC[j])

        RU = 4

        def drain(j):
            pltpu.make_async_copy(A_hbm.at[srcb[j]], bufA[j], semA[j]).wait()
            pltpu.make_async_copy(B_hbm.at[dstg[j]], bufB[j], semB[j]).wait()
            pltpu.make_async_copy(C_hbm.at[eidb[j]], bufC[j], semC[j]).wait()

            def row(r4, c2):
                for u in range(RU):
                    r = r4 * RU + u
                    for c in range(HID // LANES):
                        slc = pl.ds(c * LANES, LANES)
                        v2 = (bufA[j][r, slc] + bufB[j][r, slc]
                              + bufC[j][r, slc])
                        bufA[j][r, slc] = jnp.maximum(v2, 0.0)
                return c2

            lax.fori_loop(0, K // RU, row, 0)
            pltpu.async_copy(bufA[j], acc_sh.at[dstl[j]], ssem[j],
                             add=True)

        def superchunk(p):
            base = p * NBUF
            for j in range(NBUF):
                i = base + j
                pl.when(jnp.logical_and(p > 0, valid(i - NBUF)))(
                    lambda j=j: scat_wait(j))
                pl.when(valid(i))(lambda i=i, j=j: fire(i, j))
            for j in range(NBUF):
                pl.when(valid(base + j))(lambda j=j: drain(j))
            return p + 1

        p_end = lax.while_loop(lambda p: valid(p * NBUF), superchunk,
                               jnp.int32(0))
        for j in range(NBUF):
            pl.when(jnp.logical_and(
                p_end > 0, valid((p_end - 1) * NBUF + j)))(
                lambda j=j: scat_wait(j))
        return carry0

    lax.fori_loop(0, NSEG, segment, 0)
    plsc.subcore_barrier()
    pltpu.sync_copy(acc_sh.at[pl.ds(sid * RPT, RPT)],
                    out_hbm.at[cid, pl.ds(sid * RPT, RPT)])


_sc_call = pl.kernel(
    _sc_edge_body,
    out_type=jax.ShapeDtypeStruct((NC, APAD, HID), jnp.float32),
    mesh=plsc.VectorSubcoreMesh(core_axis_name="c", subcore_axis_name="s",
                                num_cores=NC),
    compiler_params=pltpu.CompilerParams(needs_layout_passes=False),
    scratch_types=[
        pltpu.VMEM((SEGSZ,), jnp.int32),
        pltpu.VMEM((SEGSZ,), jnp.int32),
        pltpu.VMEM((LCAP,), jnp.int32),
        pltpu.VMEM((LCAP,), jnp.int32),
        pltpu.VMEM((LCAP,), jnp.int32),
    ] + [
        t
        for _ in range(NBUF)
        for t in (
            pltpu.VMEM((K,), jnp.int32),
            pltpu.VMEM((K,), jnp.int32),
            pltpu.VMEM((K,), jnp.int32),
            pltpu.VMEM((K,), jnp.int32),
            pltpu.VMEM((K, HID), jnp.float32),
            pltpu.VMEM((K, HID), jnp.float32),
            pltpu.VMEM((K, HID), jnp.float32),
        )
    ] + [
        pltpu.VMEM_SHARED((APAD, HID), jnp.float32),
    ] + [pltpu.SemaphoreType.DMA] * (4 * NBUF),
)


def _final_body(h_ref, p_ref, Wu1, Wu2, bu, o_ref):
    agg = jnp.concatenate([p_ref[0, :HALF], p_ref[1, :HALF]], axis=0)
    o_ref[...] = (jnp.dot(h_ref[...], Wu1[...], preferred_element_type=jnp.float32)
                  + jnp.dot(agg, Wu2[...], preferred_element_type=jnp.float32)
                  + bu[...])


_final_call = pl.pallas_call(
    _final_body,
    out_shape=jax.ShapeDtypeStruct((N, LAT), jnp.float32),
)


def kernel(fn, hn, fe, edge_index, W_node, b_node, W_edge, b_edge,
           W_msg, b_msg, W_upd, b_upd):
    Wn1, Wn2 = W_node[:FN], W_node[FN:]
    W1, W2, W3 = W_msg[:HID], W_msg[HID:2 * HID], W_msg[2 * HID:]
    Wep = jnp.zeros((16, HID), jnp.float32).at[:FE].set(W_edge)
    bn = b_node.reshape(1, HID)
    be = b_edge.reshape(1, HID)
    bm = b_msg.reshape(1, HID)

    h, A, B, Wp = _prep_call(fn, hn, Wn1, Wn2, bn, W1, W2, W3, Wep, be, bm)

    fe_pad = jnp.pad(fe, ((0, 0), (0, 16 - FE)))
    C = _edgec_call(fe_pad, Wp)

    zeros = jnp.zeros((APAD, HID), jnp.float32)
    parts = _sc_call(A, B, C, edge_index[0], edge_index[1], zeros)

    Wu1, Wu2 = W_upd[:HID], W_upd[HID:]
    bu = b_upd.reshape(1, LAT)
    return _final_call(h, parts, Wu1, Wu2, bu)


# speedup vs baseline: 1.4106x; 1.4106x over previous
import jax
import jax.numpy as jnp
from jax import lax
from jax.experimental import pallas as pl
from jax.experimental.pallas import tpu as pltpu
from jax.experimental.pallas import tpu_sc as plsc

N = 10000
E = 320000
FN = 64
IN = 64
HID = 128
FE = 9
LAT = 128

NC = 2
NS = 16
LANES = 16
HALF = N // NC
EPB = E // NS
SEGSZ = 2000
NSEG = EPB // SEGSZ
SEGG = SEGSZ // LANES
K = 64
NBUF = 3
LCAP = SEGSZ + K
APAD = 5120
RPT = APAD // NS
BE = 8000


def _prep_body(fn_ref, hn_ref, Wn1, Wn2, bn, W1, W2, W3, Wep, be, bm,
               h_ref, A_ref, B_ref, Wp_ref):
    h = (jnp.dot(fn_ref[...], Wn1[...], preferred_element_type=jnp.float32)
         + jnp.dot(hn_ref[...], Wn2[...], preferred_element_type=jnp.float32)
         + bn[...])
    h_ref[...] = h
    A_ref[...] = jnp.dot(h, W1[...], preferred_element_type=jnp.float32)
    c0 = jnp.dot(be[...], W3[...], preferred_element_type=jnp.float32) + bm[...]
    B_ref[...] = jnp.dot(h, W2[...], preferred_element_type=jnp.float32) + c0
    Wp_ref[...] = jnp.dot(Wep[...], W3[...], preferred_element_type=jnp.float32)


_prep_call = pl.pallas_call(
    _prep_body,
    out_shape=[
        jax.ShapeDtypeStruct((N, HID), jnp.float32),
        jax.ShapeDtypeStruct((N, HID), jnp.float32),
        jax.ShapeDtypeStruct((N, HID), jnp.float32),
        jax.ShapeDtypeStruct((16, HID), jnp.float32),
    ],
)


def _edgec_body(fe_ref, Wp_ref, C_ref):
    C_ref[...] = jnp.dot(fe_ref[...], Wp_ref[:FE],
                         preferred_element_type=jnp.float32)


_edgec_call = pl.pallas_call(
    _edgec_body,
    grid=(E // BE,),
    in_specs=[
        pl.BlockSpec((BE, FE), lambda i: (i, 0)),
        pl.BlockSpec((16, HID), lambda i: (0, 0)),
    ],
    out_specs=pl.BlockSpec((BE, HID), lambda i: (i, 0)),
    out_shape=jax.ShapeDtypeStruct((E, HID), jnp.float32),
)


def _sc_edge_body(A_hbm, B_hbm, C_hbm, src_hbm, dst_hbm, z_hbm, out_hbm,
                  raw_s, raw_d, ls, ld, le,
                  srcb0, dstg0, dstl0, eidb0, bufA0, bufB0, bufC0,
                  srcb1, dstg1, dstl1, eidb1, bufA1, bufB1, bufC1,
                  srcb2, dstg2, dstl2, eidb2, bufA2, bufB2, bufC2,
                  acc_sh,
                  semA0, semB0, semC0, ssem0,
                  semA1, semB1, semC1, ssem1,
                  semA2, semB2, semC2, ssem2):
    srcb = (srcb0, srcb1, srcb2)
    dstg = (dstg0, dstg1, dstg2)
    dstl = (dstl0, dstl1, dstl2)
    eidb = (eidb0, eidb1, eidb2)
    bufA = (bufA0, bufA1, bufA2)
    bufB = (bufB0, bufB1, bufB2)
    bufC = (bufC0, bufC1, bufC2)
    semA = (semA0, semA1, semA2)
    semB = (semB0, semB1, semB2)
    semC = (semC0, semC1, semC2)
    ssem = (ssem0, ssem1, ssem2)
    cid = lax.axis_index("c")
    sid = lax.axis_index("s")
    lo = cid * HALF

    pltpu.sync_copy(z_hbm.at[pl.ds(sid * RPT, RPT)],
                    acc_sh.at[pl.ds(sid * RPT, RPT)])
    plsc.subcore_barrier()

    iota = lax.iota(jnp.int32, LANES)
    lo_v = jnp.full((LANES,), lo, jnp.int32)
    ones_v = jnp.full((LANES,), 1, jnp.int32)
    zeros_v = jnp.full((LANES,), 0, jnp.int32)

    def segment(seg, carry0):
        sbase = sid * EPB + seg * SEGSZ
        pltpu.sync_copy(src_hbm.at[pl.ds(sbase, SEGSZ)], raw_s)
        pltpu.sync_copy(dst_hbm.at[pl.ds(sbase, SEGSZ)], raw_d)

        def compact(g, off_v):
            s = raw_s[pl.ds(g * LANES, LANES)]
            d = raw_d[pl.ds(g * LANES, LANES)]
            rel = d - lo_v
            lm = rel.astype(jnp.uint32) < jnp.uint32(HALF)
            lmi = jnp.where(lm, ones_v, zeros_v)
            excl = plsc.cumsum(lmi) - lmi
            idxv = excl + off_v
            plsc.store_scatter(ls, [idxv], s, mask=lm)
            plsc.store_scatter(ld, [idxv], rel, mask=lm)
            plsc.store_scatter(le, [idxv],
                               jnp.full((LANES,), sbase + g * LANES,
                                        jnp.int32) + iota, mask=lm)
            return off_v + plsc.all_reduce_population_count(lm)

        cnt_v = lax.fori_loop(0, SEGG, compact, zeros_v)

        for j in range(K // LANES):
            idxv = cnt_v + jnp.full((LANES,), j * LANES, jnp.int32) + iota
            plsc.store_scatter(ls, [idxv], zeros_v)
            plsc.store_scatter(ld, [idxv],
                               jnp.full((LANES,), APAD - 1, jnp.int32))
            plsc.store_scatter(le, [idxv], zeros_v)

        def valid(i):
            return jnp.any(jnp.full((LANES,), i * K, jnp.int32) < cnt_v)

        def fire(i, j):
            for jj in range(K // LANES):
                sl = pl.ds(i * K + jj * LANES, LANES)
                t = pl.ds(jj * LANES, LANES)
                srcb[j][t] = ls[sl]
                v = ld[sl]
                dstl[j][t] = v
                dstg[j][t] = jnp.minimum(v, HALF - 1) + lo_v
                eidb[j][t] = le[sl]
            pltpu.async_copy(A_hbm.at[srcb[j]], bufA[j], semA[j])
            pltpu.async_copy(B_hbm.at[dstg[j]], bufB[j], semB[j])
            pltpu.async_copy(C_hbm.at[eidb[j]], bufC[j], semC[j])

        RU = 4

        def drain(j):
            pltpu.make_async_copy(A_hbm.at[srcb[j]], bufA[j], semA[j]).wait()
            pltpu.make_async_copy(B_hbm.at[dstg[j]], bufB[j], semB[j]).wait()
            pltpu.make_async_copy(C_hbm.at[eidb[j]], bufC[j], semC[j]).wait()

            def row(r4, c2):
                for u in range(RU):
                    r = r4 * RU + u
                    for c in range(HID // LANES):
                        slc = pl.ds(c * LANES, LANES)
                        v2 = (bufA[j][r, slc] + bufB[j][r, slc]
                              + bufC[j][r, slc])
                        bufA[j][r, slc] = jnp.maximum(v2, 0.0)
                return c2

            lax.fori_loop(0, K // RU, row, 0)
            pltpu.sync_copy(bufA[j], acc_sh.at[dstl[j]], add=True)

        def superchunk(p):
            base = p * NBUF
            for j in range(NBUF):
                pl.when(valid(base + j))(
                    lambda i=base + j, j=j: fire(i, j))
            for j in range(NBUF):
                pl.when(valid(base + j))(lambda j=j: drain(j))
            return p + 1

        lax.while_loop(lambda p: valid(p * NBUF), superchunk, jnp.int32(0))
        return carry0

    lax.fori_loop(0, NSEG, segment, 0)
    plsc.subcore_barrier()
    pltpu.sync_copy(acc_sh.at[pl.ds(sid * RPT, RPT)],
                    out_hbm.at[cid, pl.ds(sid * RPT, RPT)])


_sc_call = pl.kernel(
    _sc_edge_body,
    out_type=jax.ShapeDtypeStruct((NC, APAD, HID), jnp.float32),
    mesh=plsc.VectorSubcoreMesh(core_axis_name="c", subcore_axis_name="s",
                                num_cores=NC),
    compiler_params=pltpu.CompilerParams(needs_layout_passes=False),
    scratch_types=[
        pltpu.VMEM((SEGSZ,), jnp.int32),
        pltpu.VMEM((SEGSZ,), jnp.int32),
        pltpu.VMEM((LCAP,), jnp.int32),
        pltpu.VMEM((LCAP,), jnp.int32),
        pltpu.VMEM((LCAP,), jnp.int32),
    ] + [
        t
        for _ in range(NBUF)
        for t in (
            pltpu.VMEM((K,), jnp.int32),
            pltpu.VMEM((K,), jnp.int32),
            pltpu.VMEM((K,), jnp.int32),
            pltpu.VMEM((K,), jnp.int32),
            pltpu.VMEM((K, HID), jnp.float32),
            pltpu.VMEM((K, HID), jnp.float32),
            pltpu.VMEM((K, HID), jnp.float32),
        )
    ] + [
        pltpu.VMEM_SHARED((APAD, HID), jnp.float32),
    ] + [pltpu.SemaphoreType.DMA] * (4 * NBUF),
)


def _final_body(h_ref, p_ref, Wu1, Wu2, bu, o_ref):
    agg = jnp.concatenate([p_ref[0, :HALF], p_ref[1, :HALF]], axis=0)
    o_ref[...] = (jnp.dot(h_ref[...], Wu1[...], preferred_element_type=jnp.float32)
                  + jnp.dot(agg, Wu2[...], preferred_element_type=jnp.float32)
                  + bu[...])


_final_call = pl.pallas_call(
    _final_body,
    out_shape=jax.ShapeDtypeStruct((N, LAT), jnp.float32),
)


def kernel(fn, hn, fe, edge_index, W_node, b_node, W_edge, b_edge,
           W_msg, b_msg, W_upd, b_upd):
    Wn1, Wn2 = W_node[:FN], W_node[FN:]
    W1, W2, W3 = W_msg[:HID], W_msg[HID:2 * HID], W_msg[2 * HID:]
    Wep = jnp.zeros((16, HID), jnp.float32).at[:FE].set(W_edge)
    bn = b_node.reshape(1, HID)
    be = b_edge.reshape(1, HID)
    bm = b_msg.reshape(1, HID)

    h, A, B, Wp = _prep_call(fn, hn, Wn1, Wn2, bn, W1, W2, W3, Wep, be, bm)

    C = _edgec_call(fe, Wp)

    zeros = jnp.zeros((APAD, HID), jnp.float32)
    parts = _sc_call(A, B, C, edge_index[0], edge_index[1], zeros)

    Wu1, Wu2 = W_upd[:HID], W_upd[HID:]
    bu = b_upd.reshape(1, LAT)
    return _final_call(h, parts, Wu1, Wu2, bu)


# speedup vs baseline: 1.4613x; 1.0359x over previous
import jax
import jax.numpy as jnp
from jax import lax
from jax.experimental import pallas as pl
from jax.experimental.pallas import tpu as pltpu
from jax.experimental.pallas import tpu_sc as plsc

N = 10000
E = 320000
FN = 64
IN = 64
HID = 128
FE = 9
LAT = 128

NC = 2
NS = 16
LANES = 16
HALF = N // NC
EPB = E // NS
SEGSZ = 2000
NSEG = EPB // SEGSZ
SEGG = SEGSZ // LANES
K = 64
NBUF = 3
LCAP = SEGSZ + K
APAD = 5120
RPT = APAD // NS
BE = 8000


def _prep_body(fn_ref, hn_ref, Wn1, Wn2, bn, W1, W2, W3, Wep, be, bm,
               h_ref, A_ref, B_ref, Wp_ref):
    h = (jnp.dot(fn_ref[...], Wn1[...], preferred_element_type=jnp.float32)
         + jnp.dot(hn_ref[...], Wn2[...], preferred_element_type=jnp.float32)
         + bn[...])
    h_ref[...] = h
    A_ref[...] = jnp.dot(h, W1[...], preferred_element_type=jnp.float32)
    c0 = jnp.dot(be[...], W3[...], preferred_element_type=jnp.float32) + bm[...]
    B_ref[...] = jnp.dot(h, W2[...], preferred_element_type=jnp.float32) + c0
    Wp_ref[...] = jnp.dot(Wep[...], W3[...], preferred_element_type=jnp.float32)


_prep_call = pl.pallas_call(
    _prep_body,
    out_shape=[
        jax.ShapeDtypeStruct((N, HID), jnp.float32),
        jax.ShapeDtypeStruct((N, HID), jnp.float32),
        jax.ShapeDtypeStruct((N, HID), jnp.float32),
        jax.ShapeDtypeStruct((16, HID), jnp.float32),
    ],
)


def _edgec_body(fe_ref, Wp_ref, C_ref):
    C_ref[...] = jnp.dot(fe_ref[...], Wp_ref[:FE],
                         preferred_element_type=jnp.float32)


_edgec_call = pl.pallas_call(
    _edgec_body,
    grid=(E // BE,),
    in_specs=[
        pl.BlockSpec((BE, FE), lambda i: (i, 0)),
        pl.BlockSpec((16, HID), lambda i: (0, 0)),
    ],
    out_specs=pl.BlockSpec((BE, HID), lambda i: (i, 0)),
    out_shape=jax.ShapeDtypeStruct((E, HID), jnp.float32),
)


def _sc_edge_body(A_hbm, B_hbm, C_hbm, src_hbm, dst_hbm, z_hbm, out_hbm,
                  raw_s, raw_d, ls, ld, le,
                  srcb0, dstg0, dstl0, eidb0, bufA0, bufB0, bufC0,
                  srcb1, dstg1, dstl1, eidb1, bufA1, bufB1, bufC1,
                  srcb2, dstg2, dstl2, eidb2, bufA2, bufB2, bufC2,
                  acc_sh,
                  semA0, semB0, semC0, ssem0,
                  semA1, semB1, semC1, ssem1,
                  semA2, semB2, semC2, ssem2):
    srcb = (srcb0, srcb1, srcb2)
    dstg = (dstg0, dstg1, dstg2)
    dstl = (dstl0, dstl1, dstl2)
    eidb = (eidb0, eidb1, eidb2)
    bufA = (bufA0, bufA1, bufA2)
    bufB = (bufB0, bufB1, bufB2)
    bufC = (bufC0, bufC1, bufC2)
    semA = (semA0, semA1, semA2)
    semB = (semB0, semB1, semB2)
    semC = (semC0, semC1, semC2)
    ssem = (ssem0, ssem1, ssem2)
    cid = lax.axis_index("c")
    sid = lax.axis_index("s")
    lo = cid * HALF

    pltpu.sync_copy(z_hbm.at[pl.ds(sid * RPT, RPT)],
                    acc_sh.at[pl.ds(sid * RPT, RPT)])
    plsc.subcore_barrier()

    iota = lax.iota(jnp.int32, LANES)
    lo_v = jnp.full((LANES,), lo, jnp.int32)
    ones_v = jnp.full((LANES,), 1, jnp.int32)
    zeros_v = jnp.full((LANES,), 0, jnp.int32)

    def segment(seg, carry0):
        sbase = sid * EPB + seg * SEGSZ
        pltpu.sync_copy(src_hbm.at[pl.ds(sbase, SEGSZ)], raw_s)
        pltpu.sync_copy(dst_hbm.at[pl.ds(sbase, SEGSZ)], raw_d)

        def compact(g, off_v):
            s = raw_s[pl.ds(g * LANES, LANES)]
            d = raw_d[pl.ds(g * LANES, LANES)]
            rel = d - lo_v
            lm = rel.astype(jnp.uint32) < jnp.uint32(HALF)
            lmi = jnp.where(lm, ones_v, zeros_v)
            excl = plsc.cumsum(lmi) - lmi
            idxv = excl + off_v
            plsc.store_scatter(ls, [idxv], s, mask=lm)
            plsc.store_scatter(ld, [idxv], rel, mask=lm)
            plsc.store_scatter(le, [idxv],
                               jnp.full((LANES,), sbase + g * LANES,
                                        jnp.int32) + iota, mask=lm)
            return off_v + plsc.all_reduce_population_count(lm)

        cnt_v = lax.fori_loop(0, SEGG, compact, zeros_v)

        for j in range(K // LANES):
            idxv = cnt_v + jnp.full((LANES,), j * LANES, jnp.int32) + iota
            plsc.store_scatter(ls, [idxv], zeros_v)
            plsc.store_scatter(ld, [idxv],
                               jnp.full((LANES,), APAD - 1, jnp.int32))
            plsc.store_scatter(le, [idxv], zeros_v)

        def valid(i):
            return jnp.any(jnp.full((LANES,), i * K, jnp.int32) < cnt_v)

        def fire(i, j):
            for jj in range(K // LANES):
                sl = pl.ds(i * K + jj * LANES, LANES)
                t = pl.ds(jj * LANES, LANES)
                srcb[j][t] = ls[sl]
                v = ld[sl]
                dstl[j][t] = v
                dstg[j][t] = jnp.minimum(v, HALF - 1) + lo_v
                eidb[j][t] = le[sl]
            pltpu.async_copy(A_hbm.at[srcb[j]], bufA[j], semA[j])
            pltpu.async_copy(B_hbm.at[dstg[j]], bufB[j], semB[j])
            pltpu.async_copy(C_hbm.at[eidb[j]], bufC[j], semC[j])

        RU = 4

        def drain(j):
            pltpu.make_async_copy(A_hbm.at[srcb[j]], bufA[j], semA[j]).wait()
            pltpu.make_async_copy(B_hbm.at[dstg[j]], bufB[j], semB[j]).wait()
            pltpu.make_async_copy(C_hbm.at[eidb[j]], bufC[j], semC[j]).wait()

            def row(r4, c2):
                for u in range(RU):
                    r = r4 * RU + u
                    for c in range(HID // LANES):
                        slc = pl.ds(c * LANES, LANES)
                        v2 = (bufA[j][r, slc] + bufB[j][r, slc]
                              + bufC[j][r, slc])
                        bufA[j][r, slc] = jnp.maximum(v2, 0.0)
                return c2

            lax.fori_loop(0, K // RU, row, 0)
            pltpu.sync_copy(bufA[j], acc_sh.at[dstl[j]], add=True)

        for j in range(NBUF):
            pl.when(valid(j))(lambda j=j: fire(j, j))

        def superchunk(p):
            base = p * NBUF
            for j in range(NBUF):
                i = base + j
                pl.when(valid(i))(lambda j=j: drain(j))
                pl.when(valid(i + NBUF))(
                    lambda i=i + NBUF, j=j: fire(i, j))
            return p + 1

        lax.while_loop(lambda p: valid(p * NBUF), superchunk, jnp.int32(0))
        return carry0

    lax.fori_loop(0, NSEG, segment, 0)
    plsc.subcore_barrier()
    pltpu.sync_copy(acc_sh.at[pl.ds(sid * RPT, RPT)],
                    out_hbm.at[cid, pl.ds(sid * RPT, RPT)])


_sc_call = pl.kernel(
    _sc_edge_body,
    out_type=jax.ShapeDtypeStruct((NC, APAD, HID), jnp.float32),
    mesh=plsc.VectorSubcoreMesh(core_axis_name="c", subcore_axis_name="s",
                                num_cores=NC),
    compiler_params=pltpu.CompilerParams(needs_layout_passes=False),
    scratch_types=[
        pltpu.VMEM((SEGSZ,), jnp.int32),
        pltpu.VMEM((SEGSZ,), jnp.int32),
        pltpu.VMEM((LCAP,), jnp.int32),
        pltpu.VMEM((LCAP,), jnp.int32),
        pltpu.VMEM((LCAP,), jnp.int32),
    ] + [
        t
        for _ in range(NBUF)
        for t in (
            pltpu.VMEM((K,), jnp.int32),
            pltpu.VMEM((K,), jnp.int32),
            pltpu.VMEM((K,), jnp.int32),
            pltpu.VMEM((K,), jnp.int32),
            pltpu.VMEM((K, HID), jnp.float32),
            pltpu.VMEM((K, HID), jnp.float32),
            pltpu.VMEM((K, HID), jnp.float32),
        )
    ] + [
        pltpu.VMEM_SHARED((APAD, HID), jnp.float32),
    ] + [pltpu.SemaphoreType.DMA] * (4 * NBUF),
)


def _final_body(h_ref, p_ref, Wu1, Wu2, bu, o_ref):
    agg = jnp.concatenate([p_ref[0, :HALF], p_ref[1, :HALF]], axis=0)
    o_ref[...] = (jnp.dot(h_ref[...], Wu1[...], preferred_element_type=jnp.float32)
                  + jnp.dot(agg, Wu2[...], preferred_element_type=jnp.float32)
                  + bu[...])


_final_call = pl.pallas_call(
    _final_body,
    out_shape=jax.ShapeDtypeStruct((N, LAT), jnp.float32),
)


def kernel(fn, hn, fe, edge_index, W_node, b_node, W_edge, b_edge,
           W_msg, b_msg, W_upd, b_upd):
    Wn1, Wn2 = W_node[:FN], W_node[FN:]
    W1, W2, W3 = W_msg[:HID], W_msg[HID:2 * HID], W_msg[2 * HID:]
    Wep = jnp.zeros((16, HID), jnp.float32).at[:FE].set(W_edge)
    bn = b_node.reshape(1, HID)
    be = b_edge.reshape(1, HID)
    bm = b_msg.reshape(1, HID)

    h, A, B, Wp = _prep_call(fn, hn, Wn1, Wn2, bn, W1, W2, W3, Wep, be, bm)

    C = _edgec_call(fe, Wp)

    zeros = jnp.zeros((APAD, HID), jnp.float32)
    parts = _sc_call(A, B, C, edge_index[0], edge_index[1], zeros)

    Wu1, Wu2 = W_upd[:HID], W_upd[HID:]
    bu = b_upd.reshape(1, LAT)
    return _final_call(h, parts, Wu1, Wu2, bu)
